# baseline (speedup 1.0000x reference)
import functools

import jax
import jax.numpy as jnp
from jax import lax
from jax.experimental import pallas as pl
from jax.experimental.pallas import tpu as pltpu
from jax.experimental.pallas import tpu_sc as plsc

V = 1_000_000
F = 128
B = 4096
EPS = 1e-5

_BR = 10_000


def _row_mean_body(t_ref, o_ref):
    o_ref[:] = jnp.sum(t_ref[:], axis=1, keepdims=True) * (1.0 / F)


def _row_means(table):
    return pl.pallas_call(
        _row_mean_body,
        grid=(V // _BR,),
        in_specs=[pl.BlockSpec((_BR, F), lambda i: (i, 0))],
        out_specs=pl.BlockSpec((_BR, 1), lambda i: (i, 0)),
        out_shape=jax.ShapeDtypeStruct((V, 1), jnp.float32),
    )(table)


_NC, _NS = 2, 16
_NW = _NC * _NS
_PW = (B * F) // _NW
_CH = 128
_NCHUNK = _PW // _CH
_K = 16

@functools.cache
def _make_sc_gather():
    mesh = plsc.VectorSubcoreMesh(core_axis_name="c", subcore_axis_name="s")

    @functools.partial(
        pl.kernel,
        out_type=jax.ShapeDtypeStruct((_NW, _NCHUNK, _CH), jnp.float32),
        mesh=mesh,
        scratch_types=[
            pltpu.VMEM((_NCHUNK, _CH), jnp.int32),
            pltpu.VMEM((_NCHUNK, _CH), jnp.float32),
            pltpu.SemaphoreType.DMA,
        ],
    )
    def _sc_gather(means_hbm, idx_hbm, out_hbm, idx_v, val_v, sem):
        wid = lax.axis_index("s") * _NC + lax.axis_index("c")
        pltpu.sync_copy(idx_hbm.at[wid], idx_v)

        def body(jj, carry):
            base = jj * _K
            copies = [
                pltpu.async_copy(
                    means_hbm.at[idx_v.at[base + t]], val_v.at[base + t], sem
                )
                for t in range(_K)
            ]
            for c in copies:
                c.wait()
            return carry

        lax.fori_loop(0, _NCHUNK // _K, body, 0)
        pltpu.sync_copy(val_v, out_hbm.at[wid])

    return _sc_gather


def _head_body(p_ref, w_ref, b_ref, g_ref, be_ref, o_ref):
    p = p_ref[:]
    y = lax.dot_general(
        p, w_ref[:], (((1,), (1,)), ((), ())),
        preferred_element_type=jnp.float32,
    )
    y = y + b_ref[:]
    mu = jnp.mean(y, axis=0, keepdims=True)
    yc = y - mu
    var = jnp.mean(yc * yc, axis=0, keepdims=True)
    y = yc * lax.rsqrt(var + EPS) * g_ref[:] + be_ref[:]
    mu_r = jnp.mean(y, axis=1, keepdims=True)
    yr = y - mu_r
    var_r = jnp.mean(yr * yr, axis=1, keepdims=True)
    o_ref[:] = yr * lax.rsqrt(var_r + EPS)


def _head(pooled, W, b2, g2, be2):
    return pl.pallas_call(
        _head_body,
        out_shape=jax.ShapeDtypeStruct((B, F), jnp.float32),
    )(pooled, W, b2, g2, be2)


def kernel(x, table, W, b, gamma, beta):
    means = _row_means(table).reshape(V)
    idx = x.astype(jnp.int32).reshape(_NW, _NCHUNK, _CH)
    pooled = _make_sc_gather()(means, idx).reshape(B, F)
    return _head(
        pooled, W, b.reshape(1, F), gamma.reshape(1, F), beta.reshape(1, F)
    )
